# baseline (speedup 1.0000x reference)
import functools

import jax
import jax.numpy as jnp
from jax import lax
from jax.experimental import pallas as pl
from jax.experimental.pallas import tpu as pltpu
from jax.experimental.pallas import tpu_sc as plsc

VOCAB = 1000000
DIM = 64
B = 16384
NC = 2
NS = 16
NW = NC * NS
BPW = B // NW
CH = 128
NCHUNK = BPW // CH
LANES = 16


def _make_sc_dot():
    mesh = plsc.VectorSubcoreMesh(core_axis_name="c", subcore_axis_name="s")

    @functools.partial(
        pl.kernel,
        mesh=mesh,
        out_type=jax.ShapeDtypeStruct((B,), jnp.float32),
        scratch_types=[
            pltpu.VMEM((NCHUNK, CH), jnp.int32),
            pltpu.VMEM((NCHUNK, CH), jnp.int32),
            pltpu.VMEM((BPW,), jnp.int32),
            pltpu.VMEM((BPW,), jnp.int32),
            pltpu.VMEM((CH, 2 * DIM), jnp.float32),
            pltpu.VMEM((CH, 2 * DIM), jnp.float32),
            pltpu.VMEM((BPW,), jnp.float32),
            pltpu.VMEM((LANES, LANES + 1), jnp.float32),
            pltpu.SemaphoreType.DMA,
            pltpu.SemaphoreType.DMA,
        ],
        compiler_params=pltpu.CompilerParams(needs_layout_passes=False),
    )
    def sc_dot(slab1_hbm, slab2_hbm, par1_hbm, par2_hbm, emb_hbm, ctx_hbm,
               out_hbm, slab1_v, slab2_v, par1_v, par2_v, rows1_v, rows2_v,
               dot_v, pbuf_v, sem1, sem2):
        wid = lax.axis_index("s") * NC + lax.axis_index("c")
        base = wid * BPW
        pltpu.sync_copy(slab1_hbm.at[wid], slab1_v)
        pltpu.sync_copy(slab2_hbm.at[wid], slab2_v)
        pltpu.sync_copy(par1_hbm.at[pl.ds(base, BPW)], par1_v)
        pltpu.sync_copy(par2_hbm.at[pl.ds(base, BPW)], par2_v)
        lane = jnp.arange(LANES, dtype=jnp.int32)

        def chunk(c, carry):
            cbase = c * CH
            c1 = pltpu.async_copy(emb_hbm.at[slab1_v.at[c]], rows1_v, sem1)
            c2 = pltpu.async_copy(ctx_hbm.at[slab2_v.at[c]], rows2_v, sem2)
            c1.wait()
            c2.wait()
            for g in range(CH // LANES):
                pv1 = par1_v[pl.ds(cbase + g * LANES, LANES)]
                pv2 = par2_v[pl.ds(cbase + g * LANES, LANES)]
                for j in range(LANES):
                    i = g * LANES + j
                    o1 = pv1[j] * DIM
                    o2 = pv2[j] * DIM
                    acc = (rows1_v[i, pl.ds(o1, LANES)]
                           * rows2_v[i, pl.ds(o2, LANES)])
                    for k in range(1, DIM // LANES):
                        acc = acc + (rows1_v[i, pl.ds(o1 + k * LANES, LANES)]
                                     * rows2_v[i, pl.ds(o2 + k * LANES, LANES)])
                    pbuf_v[j, pl.ds(0, LANES)] = acc
                dot = jnp.zeros((LANES,), jnp.float32)
                for k in range(LANES):
                    col = plsc.load_gather(
                        pbuf_v, [lane, jnp.full((LANES,), k, jnp.int32)])
                    dot = dot + col
                dot_v[pl.ds(cbase + g * LANES, LANES)] = dot
            return carry

        lax.fori_loop(0, NCHUNK, chunk, 0)
        pltpu.sync_copy(dot_v, out_hbm.at[pl.ds(base, BPW)])

    return sc_dot


_SC_DOT = _make_sc_dot()


def _tc_loss_body(dot_ref, tgt_ref, out_ref):
    x = jax.nn.sigmoid(dot_ref[...])
    t = tgt_ref[...]
    l = jnp.clip(x, 0.0, None) - x * t + jnp.log1p(jnp.exp(-jnp.abs(x)))
    out_ref[...] = (jnp.sum(l) * (1.0 / B)).reshape(1, 1)


_TC_LOSS = pl.pallas_call(
    _tc_loss_body,
    out_shape=jax.ShapeDtypeStruct((1, 1), jnp.float32),
)


def kernel(word1_index, word2_index, target, emb_table, ctx_table):
    idx1 = word1_index.astype(jnp.int32)
    idx2 = word2_index.astype(jnp.int32)
    slab1 = (idx1 >> 1).reshape(NW, NCHUNK, CH)
    slab2 = (idx2 >> 1).reshape(NW, NCHUNK, CH)
    par1 = idx1 & 1
    par2 = idx2 & 1
    emb2 = emb_table.reshape(VOCAB // 2, 2 * DIM)
    ctx2 = ctx_table.reshape(VOCAB // 2, 2 * DIM)
    dot = _SC_DOT(slab1, slab2, par1, par2, emb2, ctx2)
    loss = _TC_LOSS(dot.reshape(128, 128), target.reshape(128, 128))
    return loss[0, 0]


# speedup vs baseline: 1.7659x; 1.7659x over previous
import functools

import jax
import jax.numpy as jnp
from jax import lax
from jax.experimental import pallas as pl
from jax.experimental.pallas import tpu as pltpu
from jax.experimental.pallas import tpu_sc as plsc

VOCAB = 1000000
DIM = 64
B = 16384
NC = 2
NS = 16
NW = NC * NS
LANES = 16
WIN = 512
NFULL = VOCAB // WIN
TAIL0 = NFULL * WIN
ROUNDS = NFULL // NW
NB = 2048
OUTR = B + NW


def _make_sc_gather():
    mesh = plsc.VectorSubcoreMesh(core_axis_name="c", subcore_axis_name="s")

    @functools.partial(
        pl.kernel,
        mesh=mesh,
        out_type=jax.ShapeDtypeStruct((OUTR, 2 * DIM), jnp.float32),
        scratch_types=[
            pltpu.VMEM((B + LANES,), jnp.int32),
            pltpu.VMEM((B + LANES,), jnp.int32),
            pltpu.VMEM((NB,), jnp.int32),
            pltpu.VMEM((DIM, WIN), jnp.float32),
            pltpu.VMEM((DIM, DIM), jnp.float32),
            pltpu.VMEM((LANES, 2 * DIM), jnp.float32),
            pltpu.SemaphoreType.DMA,
            pltpu.SemaphoreType.DMA,
        ],
        compiler_params=pltpu.CompilerParams(needs_layout_passes=False),
    )
    def sc_gather(sidx_hbm, sord_hbm, bnds_hbm, tabT_hbm, out_hbm,
                  sidx_v, sord_v, bnds_v, win_v, tail_v, stg_v, sem, semo):
        wid = lax.axis_index("s") * NC + lax.axis_index("c")
        pltpu.sync_copy(sidx_hbm, sidx_v.at[pl.ds(0, B)])
        pltpu.sync_copy(sord_hbm, sord_v.at[pl.ds(0, B)])
        pltpu.sync_copy(bnds_hbm, bnds_v)
        lane = jnp.arange(LANES, dtype=jnp.int32)
        zero16 = jnp.zeros((LANES,), jnp.float32)
        for j in range(LANES):
            for k in range(DIM // LANES):
                stg_v[j, pl.ds(DIM + k * LANES, LANES)] = zero16
        dummy = B + wid

        def do_window(w, w0, wbuf, wlen):
            bv = bnds_v[pl.ds(w, LANES)]
            lo, hi = bv[0], bv[1]
            ngroups = (hi - lo + LANES - 1) // LANES

            def group(g, carry):
                q = lo + g * LANES
                iv = sidx_v[pl.ds(q, LANES)]
                pv = sord_v[pl.ds(q, LANES)]
                cv = iv - w0
                nvalid = hi - q
                for jj in range(LANES):
                    @pl.when(jj < nvalid)
                    def _():
                        c = jnp.broadcast_to(cv[jj], (LANES,))
                        for k in range(DIM // LANES):
                            col = plsc.load_gather(
                                wbuf, [lane + k * LANES, c])
                            stg_v[jj, pl.ds(k * LANES, LANES)] = col
                pv_m = jnp.where(lane < nvalid, pv, dummy)
                pltpu.async_copy(stg_v, out_hbm.at[pv_m], semo).wait()
                return carry

            lax.fori_loop(0, ngroups, group, 0)

        def round_body(r, carry):
            w = r * NW + wid
            w0 = w * WIN
            pltpu.sync_copy(tabT_hbm.at[:, pl.ds(w0, WIN)], win_v)
            do_window(w, w0, win_v, WIN)
            return carry

        lax.fori_loop(0, ROUNDS, round_body, 0)

        @pl.when(wid == 0)
        def _():
            w = NFULL - 1
            pltpu.sync_copy(tabT_hbm.at[:, pl.ds(w * WIN, WIN)], win_v)
            do_window(w, w * WIN, win_v, WIN)

        @pl.when(wid == 1)
        def _():
            pltpu.sync_copy(tabT_hbm.at[:, pl.ds(TAIL0, DIM)], tail_v)
            do_window(NFULL, TAIL0, tail_v, DIM)

    return sc_gather


_SC_GATHER = _make_sc_gather()


def _tc_loss_body(g1_ref, g2_ref, tgt_ref, out_ref):
    prod = g1_ref[...] * g2_ref[...]
    ones = jnp.ones((2 * DIM, 1), jnp.float32)
    d = jax.lax.dot_general(prod, ones, (((1,), (0,)), ((), ())),
                            preferred_element_type=jnp.float32)
    x = jax.nn.sigmoid(d)
    t = tgt_ref[...]
    l = jnp.clip(x, 0.0, None) - x * t + jnp.log1p(jnp.exp(-jnp.abs(x)))
    mask = jax.lax.broadcasted_iota(jnp.int32, (OUTR, 1), 0) < B
    out_ref[...] = (jnp.sum(jnp.where(mask, l, 0.0)) * (1.0 / B)).reshape(1, 1)


_TC_LOSS = pl.pallas_call(
    _tc_loss_body,
    out_shape=jax.ShapeDtypeStruct((1, 1), jnp.float32),
)


def _prep(idx):
    sidx = jnp.sort(idx)
    sord = jnp.argsort(idx).astype(jnp.int32)
    starts = jnp.arange(NB, dtype=jnp.int32) * WIN
    bnds = jnp.searchsorted(sidx, starts).astype(jnp.int32)
    return sidx, sord, bnds


def kernel(word1_index, word2_index, target, emb_table, ctx_table):
    idx1 = word1_index.astype(jnp.int32)
    idx2 = word2_index.astype(jnp.int32)
    s1, o1, b1 = _prep(idx1)
    s2, o2, b2 = _prep(idx2)
    g1 = _SC_GATHER(s1, o1, b1, emb_table.T)
    g2 = _SC_GATHER(s2, o2, b2, ctx_table.T)
    tpad = jnp.concatenate(
        [target, jnp.zeros((NW,), jnp.float32)]).reshape(OUTR, 1)
    loss = _TC_LOSS(g1, g2, tpad)
    return loss[0, 0]


# speedup vs baseline: 3.0037x; 1.7010x over previous
import functools

import jax
import jax.numpy as jnp
from jax import lax
from jax.experimental import pallas as pl
from jax.experimental.pallas import tpu as pltpu
from jax.experimental.pallas import tpu_sc as plsc

VOCAB = 1000000
DIM = 64
B = 16384
NC = 2
NS = 16
NW = NC * NS
LANES = 16
WIN = 512
NFULL = VOCAB // WIN
TAIL0 = NFULL * WIN
ROUNDS = NFULL // NW
OUTR = B + NW


def _make_sc_gather():
    mesh = plsc.VectorSubcoreMesh(core_axis_name="c", subcore_axis_name="s")

    @functools.partial(
        pl.kernel,
        mesh=mesh,
        out_type=jax.ShapeDtypeStruct((OUTR, 2 * DIM), jnp.float32),
        scratch_types=[
            pltpu.VMEM((B + LANES,), jnp.int32),
            pltpu.VMEM((B + LANES,), jnp.int32),
            pltpu.VMEM((2, DIM, WIN), jnp.float32),
            pltpu.VMEM((DIM, DIM), jnp.float32),
            pltpu.VMEM((LANES, 2 * DIM), jnp.float32),
            pltpu.SemaphoreType.DMA,
            pltpu.SemaphoreType.DMA,
            pltpu.SemaphoreType.DMA,
        ],
        compiler_params=pltpu.CompilerParams(needs_layout_passes=False),
    )
    def sc_gather(sidx_hbm, sord_hbm, tabT_hbm, out_hbm,
                  sidx_v, sord_v, win_v, tail_v, stg_v, semA, semB, semo):
        wid = lax.axis_index("s") * NC + lax.axis_index("c")
        pltpu.sync_copy(sidx_hbm, sidx_v.at[pl.ds(0, B)])
        pltpu.sync_copy(sord_hbm, sord_v.at[pl.ds(0, B)])
        lane = jnp.arange(LANES, dtype=jnp.int32)
        zero16 = jnp.zeros((LANES,), jnp.float32)
        for j in range(LANES):
            for k in range(DIM // LANES):
                stg_v[j, pl.ds(DIM + k * LANES, LANES)] = zero16
        dummy = B + wid

        def lower_bound(v):
            def step(it, lohi):
                lo, hi = lohi
                mid = (lo + hi) >> 1
                val = sidx_v[pl.ds(mid, LANES)][0]
                smaller = val < v
                return (jnp.where(smaller, mid + 1, lo),
                        jnp.where(smaller, hi, mid))
            lo, _ = lax.fori_loop(
                0, 14, step, (jnp.int32(0), jnp.int32(B)))
            return lo

        def do_window(w0, wbuf, wlen):
            lo = lower_bound(w0)
            hi = lower_bound(w0 + wlen)
            ngroups = (hi - lo + LANES - 1) // LANES

            def group(g, carry):
                q = lo + g * LANES
                iv = sidx_v[pl.ds(q, LANES)]
                pv = sord_v[pl.ds(q, LANES)]
                cv = iv - w0
                nvalid = hi - q
                for jj in range(LANES):
                    @pl.when(jj < nvalid)
                    def _():
                        c = jnp.broadcast_to(cv[jj], (LANES,))
                        for k in range(DIM // LANES):
                            col = plsc.load_gather(
                                wbuf, [lane + k * LANES, c])
                            stg_v[jj, pl.ds(k * LANES, LANES)] = col
                pv_m = jnp.where(lane < nvalid, pv, dummy)
                pltpu.async_copy(stg_v, out_hbm.at[pv_m], semo).wait()
                return carry

            lax.fori_loop(0, ngroups, group, 0)

        def start(r, buf, sem):
            w0 = (r * NW + wid) * WIN
            pltpu.async_copy(tabT_hbm.at[:, pl.ds(w0, WIN)], buf, sem)

        def drain(buf, sem):
            pltpu.make_async_copy(
                tabT_hbm.at[:, pl.ds(0, WIN)], buf, sem).wait()

        def process(r, buf):
            do_window((r * NW + wid) * WIN, buf, WIN)

        start(0, win_v.at[0], semA)

        def body2(rr, carry):
            r0 = rr * 2

            @pl.when(r0 + 1 < ROUNDS)
            def _():
                start(r0 + 1, win_v.at[1], semB)
            drain(win_v.at[0], semA)
            process(r0, win_v.at[0])

            @pl.when(r0 + 2 < ROUNDS)
            def _():
                start(r0 + 2, win_v.at[0], semA)

            @pl.when(r0 + 1 < ROUNDS)
            def _():
                drain(win_v.at[1], semB)
                process(r0 + 1, win_v.at[1])
            return carry

        lax.fori_loop(0, (ROUNDS + 1) // 2, body2, 0)

        @pl.when(wid == 0)
        def _():
            w0 = (NFULL - 1) * WIN
            pltpu.sync_copy(tabT_hbm.at[:, pl.ds(w0, WIN)], win_v.at[0])
            do_window(w0, win_v.at[0], WIN)

        @pl.when(wid == 1)
        def _():
            pltpu.sync_copy(tabT_hbm.at[:, pl.ds(TAIL0, DIM)], tail_v)
            do_window(TAIL0, tail_v, DIM)

    return sc_gather


_SC_GATHER = _make_sc_gather()


def _tc_loss_body(g1_ref, g2_ref, tgt_ref, out_ref):
    prod = g1_ref[...] * g2_ref[...]
    ones = jnp.ones((2 * DIM, 1), jnp.float32)
    d = jax.lax.dot_general(prod, ones, (((1,), (0,)), ((), ())),
                            preferred_element_type=jnp.float32)
    x = jax.nn.sigmoid(d)
    t = tgt_ref[...]
    l = jnp.clip(x, 0.0, None) - x * t + jnp.log1p(jnp.exp(-jnp.abs(x)))
    mask = jax.lax.broadcasted_iota(jnp.int32, (OUTR, 1), 0) < B
    out_ref[...] = (jnp.sum(jnp.where(mask, l, 0.0)) * (1.0 / B)).reshape(1, 1)


_TC_LOSS = pl.pallas_call(
    _tc_loss_body,
    out_shape=jax.ShapeDtypeStruct((1, 1), jnp.float32),
)


def _prep(idx):
    sord = jnp.argsort(idx).astype(jnp.int32)
    sidx = idx[sord]
    return sidx, sord


def kernel(word1_index, word2_index, target, emb_table, ctx_table):
    idx1 = word1_index.astype(jnp.int32)
    idx2 = word2_index.astype(jnp.int32)
    s1, o1 = _prep(idx1)
    s2, o2 = _prep(idx2)
    g1 = _SC_GATHER(s1, o1, emb_table.T)
    g2 = _SC_GATHER(s2, o2, ctx_table.T)
    tpad = jnp.concatenate(
        [target, jnp.zeros((NW,), jnp.float32)]).reshape(OUTR, 1)
    loss = _TC_LOSS(g1, g2, tpad)
    return loss[0, 0]


# speedup vs baseline: 3.0185x; 1.0049x over previous
import functools

import jax
import jax.numpy as jnp
from jax import lax
from jax.experimental import pallas as pl
from jax.experimental.pallas import tpu as pltpu
from jax.experimental.pallas import tpu_sc as plsc

VOCAB = 1000000
DIM = 64
B = 16384
NC = 2
NS = 16
NW = NC * NS
LANES = 16
WIN = 512
NFULL = VOCAB // WIN
TAIL0 = NFULL * WIN
ROUNDS = NFULL // NW
OUTR = B + NW


def _make_sc_gather():
    mesh = plsc.VectorSubcoreMesh(core_axis_name="c", subcore_axis_name="s")

    @functools.partial(
        pl.kernel,
        mesh=mesh,
        out_type=jax.ShapeDtypeStruct((OUTR, 2 * DIM), jnp.float32),
        scratch_types=[
            pltpu.VMEM((B + LANES,), jnp.int32),
            pltpu.VMEM((B + LANES,), jnp.int32),
            pltpu.VMEM((2, DIM, WIN), jnp.float32),
            pltpu.VMEM((DIM, DIM), jnp.float32),
            pltpu.VMEM((LANES, 2 * DIM), jnp.float32),
            pltpu.VMEM((LANES * 65,), jnp.float32),
            pltpu.SemaphoreType.DMA,
            pltpu.SemaphoreType.DMA,
            pltpu.SemaphoreType.DMA,
        ],
        compiler_params=pltpu.CompilerParams(needs_layout_passes=False),
    )
    def sc_gather(sidx_hbm, sord_hbm, tabT_hbm, out_hbm,
                  sidx_v, sord_v, win_v, tail_v, stg_v, tr_v,
                  semA, semB, semo):
        wid = lax.axis_index("s") * NC + lax.axis_index("c")
        pltpu.sync_copy(sidx_hbm, sidx_v.at[pl.ds(0, B)])
        pltpu.sync_copy(sord_hbm, sord_v.at[pl.ds(0, B)])
        lane = jnp.arange(LANES, dtype=jnp.int32)
        zero16 = jnp.zeros((LANES,), jnp.float32)
        for j in range(LANES):
            for k in range(DIM // LANES):
                stg_v[j, pl.ds(DIM + k * LANES, LANES)] = zero16
        dummy = B + wid

        def lower_bound(v):
            def step(it, lohi):
                lo, hi = lohi
                mid = (lo + hi) >> 1
                val = sidx_v[pl.ds(mid, LANES)][0]
                smaller = val < v
                return (jnp.where(smaller, mid + 1, lo),
                        jnp.where(smaller, hi, mid))
            lo, _ = lax.fori_loop(
                0, 14, step, (jnp.int32(0), jnp.int32(B)))
            return lo

        def drain_scatter():
            pltpu.make_async_copy(
                out_hbm.at[pl.ds(0, LANES)], stg_v, semo).wait()

        def do_window(w0, wbuf, wlen):
            lo = lower_bound(w0)
            hi = lower_bound(w0 + wlen)
            ngroups = (hi - lo + LANES - 1) // LANES

            def group(g, carry):
                q = lo + g * LANES
                iv = sidx_v[pl.ds(q, LANES)]
                pv = sord_v[pl.ds(q, LANES)]
                nvalid = hi - q
                valid = lane < nvalid
                cv = jnp.where(valid, iv - w0, 0)
                for f in range(DIM):
                    colf = plsc.load_gather(
                        wbuf, [jnp.full((LANES,), f, jnp.int32), cv])
                    plsc.store_scatter(tr_v, [lane * 65 + f], colf)

                @pl.when(g >= 1)
                def _():
                    drain_scatter()
                for jj in range(LANES):
                    for k in range(DIM // LANES):
                        row = plsc.load_gather(
                            tr_v, [jj * 65 + k * LANES + lane])
                        stg_v[jj, pl.ds(k * LANES, LANES)] = row
                pv_m = jnp.where(valid, pv, dummy)
                pltpu.async_copy(stg_v, out_hbm.at[pv_m], semo)
                return carry

            lax.fori_loop(0, ngroups, group, 0)

            @pl.when(ngroups >= 1)
            def _():
                drain_scatter()

        def start(r, buf, sem):
            w0 = (r * NW + wid) * WIN
            pltpu.async_copy(tabT_hbm.at[:, pl.ds(w0, WIN)], buf, sem)

        def drain(buf, sem):
            pltpu.make_async_copy(
                tabT_hbm.at[:, pl.ds(0, WIN)], buf, sem).wait()

        def process(r, buf):
            do_window((r * NW + wid) * WIN, buf, WIN)

        start(0, win_v.at[0], semA)

        def body2(rr, carry):
            r0 = rr * 2

            @pl.when(r0 + 1 < ROUNDS)
            def _():
                start(r0 + 1, win_v.at[1], semB)
            drain(win_v.at[0], semA)
            process(r0, win_v.at[0])

            @pl.when(r0 + 2 < ROUNDS)
            def _():
                start(r0 + 2, win_v.at[0], semA)

            @pl.when(r0 + 1 < ROUNDS)
            def _():
                drain(win_v.at[1], semB)
                process(r0 + 1, win_v.at[1])
            return carry

        lax.fori_loop(0, (ROUNDS + 1) // 2, body2, 0)

        @pl.when(wid == 0)
        def _():
            w0 = (NFULL - 1) * WIN
            pltpu.sync_copy(tabT_hbm.at[:, pl.ds(w0, WIN)], win_v.at[0])
            do_window(w0, win_v.at[0], WIN)

        @pl.when(wid == 1)
        def _():
            pltpu.sync_copy(tabT_hbm.at[:, pl.ds(TAIL0, DIM)], tail_v)
            do_window(TAIL0, tail_v, DIM)

    return sc_gather


_SC_GATHER = _make_sc_gather()


def _tc_loss_body(g1_ref, g2_ref, tgt_ref, out_ref):
    prod = g1_ref[...] * g2_ref[...]
    ones = jnp.ones((2 * DIM, 1), jnp.float32)
    d = jax.lax.dot_general(prod, ones, (((1,), (0,)), ((), ())),
                            preferred_element_type=jnp.float32)
    x = jax.nn.sigmoid(d)
    t = tgt_ref[...]
    l = jnp.clip(x, 0.0, None) - x * t + jnp.log1p(jnp.exp(-jnp.abs(x)))
    mask = jax.lax.broadcasted_iota(jnp.int32, (OUTR, 1), 0) < B
    out_ref[...] = (jnp.sum(jnp.where(mask, l, 0.0)) * (1.0 / B)).reshape(1, 1)


_TC_LOSS = pl.pallas_call(
    _tc_loss_body,
    out_shape=jax.ShapeDtypeStruct((1, 1), jnp.float32),
)


def _prep(idx):
    sord = jnp.argsort(idx).astype(jnp.int32)
    sidx = idx[sord]
    return sidx, sord


def kernel(word1_index, word2_index, target, emb_table, ctx_table):
    idx1 = word1_index.astype(jnp.int32)
    idx2 = word2_index.astype(jnp.int32)
    s1, o1 = _prep(idx1)
    s2, o2 = _prep(idx2)
    g1 = _SC_GATHER(s1, o1, emb_table.T)
    g2 = _SC_GATHER(s2, o2, ctx_table.T)
    tpad = jnp.concatenate(
        [target, jnp.zeros((NW,), jnp.float32)]).reshape(OUTR, 1)
    loss = _TC_LOSS(g1, g2, tpad)
    return loss[0, 0]


# speedup vs baseline: 3.0903x; 1.0238x over previous
import functools

import jax
import jax.numpy as jnp
from jax import lax
from jax.experimental import pallas as pl
from jax.experimental.pallas import tpu as pltpu
from jax.experimental.pallas import tpu_sc as plsc

VOCAB = 1000000
DIM = 64
B = 16384
NC = 2
NS = 16
NW = NC * NS
LANES = 16
WIN = 512
NFULL = VOCAB // WIN
TAIL0 = NFULL * WIN
ROUNDS = NFULL // NW
OUTR = B + NW


def _make_sc_gather():
    mesh = plsc.VectorSubcoreMesh(core_axis_name="c", subcore_axis_name="s")

    @functools.partial(
        pl.kernel,
        mesh=mesh,
        out_type=(jax.ShapeDtypeStruct((OUTR, 2 * DIM), jnp.float32),
                  jax.ShapeDtypeStruct((OUTR, 2 * DIM), jnp.float32)),
        scratch_types=[
            pltpu.VMEM((B + LANES,), jnp.int32),
            pltpu.VMEM((B + LANES,), jnp.int32),
            pltpu.VMEM((2, DIM, WIN), jnp.float32),
            pltpu.VMEM((DIM, DIM), jnp.float32),
            pltpu.VMEM((LANES, 2 * DIM), jnp.float32),
            pltpu.VMEM((LANES * 65,), jnp.float32),
            pltpu.SemaphoreType.DMA,
            pltpu.SemaphoreType.DMA,
            pltpu.SemaphoreType.DMA,
        ],
        compiler_params=pltpu.CompilerParams(needs_layout_passes=False),
    )
    def sc_gather(s1_hbm, o1_hbm, s2_hbm, o2_hbm, tab1_hbm, tab2_hbm,
                  out1_hbm, out2_hbm, sidx_v, sord_v, win_v, tail_v,
                  stg_v, tr_v, semA, semB, semo):
        wid = lax.axis_index("s") * NC + lax.axis_index("c")
        lane = jnp.arange(LANES, dtype=jnp.int32)
        zero16 = jnp.zeros((LANES,), jnp.float32)
        for j in range(LANES):
            for k in range(DIM // LANES):
                stg_v[j, pl.ds(DIM + k * LANES, LANES)] = zero16
        dummy = B + wid

        def lower_bound(v):
            def step(it, lohi):
                lo, hi = lohi
                mid = (lo + hi) >> 1
                val = sidx_v[pl.ds(mid, LANES)][0]
                smaller = val < v
                return (jnp.where(smaller, mid + 1, lo),
                        jnp.where(smaller, hi, mid))
            lo, _ = lax.fori_loop(
                0, 14, step, (jnp.int32(0), jnp.int32(B)))
            return lo

        def phase(tabT_hbm, out_hbm):
            def drain_scatter():
                pltpu.make_async_copy(
                    out_hbm.at[pl.ds(0, LANES)], stg_v, semo).wait()

            def do_window(w0, wbuf, wlen):
                lo = lower_bound(w0)
                hi = lower_bound(w0 + wlen)
                ngroups = (hi - lo + LANES - 1) // LANES

                def group(g, carry):
                    q = lo + g * LANES
                    iv = sidx_v[pl.ds(q, LANES)]
                    pv = sord_v[pl.ds(q, LANES)]
                    nvalid = hi - q
                    valid = lane < nvalid
                    cv = jnp.where(valid, iv - w0, 0)
                    for f in range(DIM):
                        colf = plsc.load_gather(
                            wbuf, [jnp.full((LANES,), f, jnp.int32), cv])
                        plsc.store_scatter(tr_v, [lane * 65 + f], colf)

                    @pl.when(g >= 1)
                    def _():
                        drain_scatter()
                    for jj in range(LANES):
                        for k in range(DIM // LANES):
                            row = plsc.load_gather(
                                tr_v, [jj * 65 + k * LANES + lane])
                            stg_v[jj, pl.ds(k * LANES, LANES)] = row
                    pv_m = jnp.where(valid, pv, dummy)
                    pltpu.async_copy(stg_v, out_hbm.at[pv_m], semo)
                    return carry

                lax.fori_loop(0, ngroups, group, 0)

                @pl.when(ngroups >= 1)
                def _():
                    drain_scatter()

            def start(r, buf, sem):
                w0 = (r * NW + wid) * WIN
                pltpu.async_copy(tabT_hbm.at[:, pl.ds(w0, WIN)], buf, sem)

            def drain_win(buf, sem):
                pltpu.make_async_copy(
                    tabT_hbm.at[:, pl.ds(0, WIN)], buf, sem).wait()

            def process(r, buf):
                do_window((r * NW + wid) * WIN, buf, WIN)

            start(0, win_v.at[0], semA)

            def body2(rr, carry):
                r0 = rr * 2

                @pl.when(r0 + 1 < ROUNDS)
                def _():
                    start(r0 + 1, win_v.at[1], semB)
                drain_win(win_v.at[0], semA)
                process(r0, win_v.at[0])

                @pl.when(r0 + 2 < ROUNDS)
                def _():
                    start(r0 + 2, win_v.at[0], semA)

                @pl.when(r0 + 1 < ROUNDS)
                def _():
                    drain_win(win_v.at[1], semB)
                    process(r0 + 1, win_v.at[1])
                return carry

            lax.fori_loop(0, (ROUNDS + 1) // 2, body2, 0)

            @pl.when(wid == 0)
            def _():
                w0 = (NFULL - 1) * WIN
                pltpu.sync_copy(tabT_hbm.at[:, pl.ds(w0, WIN)], win_v.at[0])
                do_window(w0, win_v.at[0], WIN)

            @pl.when(wid == 1)
            def _():
                pltpu.sync_copy(tabT_hbm.at[:, pl.ds(TAIL0, DIM)], tail_v)
                do_window(TAIL0, tail_v, DIM)

        pltpu.sync_copy(s1_hbm, sidx_v.at[pl.ds(0, B)])
        pltpu.sync_copy(o1_hbm, sord_v.at[pl.ds(0, B)])
        phase(tab1_hbm, out1_hbm)
        pltpu.sync_copy(s2_hbm, sidx_v.at[pl.ds(0, B)])
        pltpu.sync_copy(o2_hbm, sord_v.at[pl.ds(0, B)])
        phase(tab2_hbm, out2_hbm)

    return sc_gather


_SC_GATHER = _make_sc_gather()


def _tc_loss_body(g1_ref, g2_ref, tgt_ref, out_ref):
    prod = g1_ref[...] * g2_ref[...]
    ones = jnp.ones((2 * DIM, 1), jnp.float32)
    d = jax.lax.dot_general(prod, ones, (((1,), (0,)), ((), ())),
                            preferred_element_type=jnp.float32)
    x = jax.nn.sigmoid(d)
    t = tgt_ref[...]
    l = jnp.clip(x, 0.0, None) - x * t + jnp.log1p(jnp.exp(-jnp.abs(x)))
    mask = jax.lax.broadcasted_iota(jnp.int32, (OUTR, 1), 0) < B
    out_ref[...] = (jnp.sum(jnp.where(mask, l, 0.0)) * (1.0 / B)).reshape(1, 1)


_TC_LOSS = pl.pallas_call(
    _tc_loss_body,
    out_shape=jax.ShapeDtypeStruct((1, 1), jnp.float32),
)


def _prep(idx):
    sord = jnp.argsort(idx).astype(jnp.int32)
    sidx = idx[sord]
    return sidx, sord


def kernel(word1_index, word2_index, target, emb_table, ctx_table):
    idx1 = word1_index.astype(jnp.int32)
    idx2 = word2_index.astype(jnp.int32)
    s1, o1 = _prep(idx1)
    s2, o2 = _prep(idx2)
    g1, g2 = _SC_GATHER(s1, o1, s2, o2, emb_table.T, ctx_table.T)
    tpad = jnp.concatenate(
        [target, jnp.zeros((NW,), jnp.float32)]).reshape(OUTR, 1)
    loss = _TC_LOSS(g1, g2, tpad)
    return loss[0, 0]
